# baseline (speedup 1.0000x reference)
import functools

import jax
import jax.numpy as jnp
from jax.experimental import pallas as pl
from jax.experimental.pallas import tpu as pltpu

B, S, A, F, H = 128, 50, 16, 16, 64


def _fused(x_ref, na_ref, wih_ref, whh_ref, bias_ref,
           wp1_ref, bp1_ref, ws1_ref, wn1_ref, b1_ref,
           wp2_ref, bp2_ref, ws2_ref, wn2_ref, b2_ref,
           out_ref):
    N = A * B
    wih = wih_ref[...]
    whh = whh_ref[...]
    bias = bias_ref[...]

    UNROLL = 10

    def step(j, carry):
        h, c = carry
        half = jnp.bfloat16(0.5)
        for k in range(UNROLL):
            t = j * UNROLL + k
            xt = x_ref[t]
            gates = (jnp.dot(wih, xt, preferred_element_type=jnp.float32)
                     + jnp.dot(whh, h, preferred_element_type=jnp.float32)
                     ).astype(jnp.bfloat16) + bias
            t4 = jnp.tanh(gates)
            i = half * t4[0 * H:1 * H] + half
            f = half * t4[1 * H:2 * H] + half
            g = t4[2 * H:3 * H]
            o = half * t4[3 * H:4 * H] + half
            c = f * c + i * g
            h = o * jnp.tanh(c)
        return (h, c)

    h0 = jnp.zeros((H, N), jnp.bfloat16)
    c0 = jnp.zeros((H, N), jnp.bfloat16)
    hbf, _ = jax.lax.fori_loop(0, S // UNROLL, step, (h0, c0))
    hn = hbf.astype(jnp.float32)

    na = na_ref[...]

    def sage(hin, wp, bp, ws, wn, bb):
        m = jnp.maximum(jnp.dot(wp, hin, preferred_element_type=jnp.float32) + bp, 0.0)
        mv = [jnp.where(na > float(a), m[:, a * B:(a + 1) * B], -1.0)
              for a in range(A)]
        m1 = functools.reduce(jnp.maximum, mv)
        cnt = functools.reduce(
            jnp.add, [(v == m1).astype(jnp.float32) for v in mv])
        m2 = functools.reduce(
            jnp.maximum, [jnp.where(v == m1, -1.0, v) for v in mv])
        unique = cnt == 1.0
        agg = jnp.concatenate(
            [jnp.where((v == m1) & unique, m2, m1) for v in mv], axis=1)
        return (jnp.dot(ws, hin, preferred_element_type=jnp.float32)
                + jnp.dot(wn, agg, preferred_element_type=jnp.float32)
                + bb)

    h1 = jnp.tanh(sage(hn, wp1_ref[...], bp1_ref[...], ws1_ref[...],
                       wn1_ref[...], b1_ref[...]))
    h2 = sage(h1, wp2_ref[...], bp2_ref[...], ws2_ref[...],
              wn2_ref[...], b2_ref[...])

    pooled = functools.reduce(
        jnp.add, [jnp.where(na > float(a), h2[:, a * B:(a + 1) * B], 0.0)
                  for a in range(A)])
    out_ref[...] = pooled / na


def kernel(agent_obs, hideout_obs, timestep_obs, num_agents,
           W_ih, W_hh, b_ih, b_hh,
           Wpool1, bpool1, Wself1, Wneigh1, b1,
           Wpool2, bpool2, Wself2, Wneigh2, b2):
    x = jnp.transpose(agent_obs, (1, 3, 2, 0)).reshape(S, F, A * B)
    x = x.astype(jnp.bfloat16)
    na = num_agents.astype(jnp.float32).reshape(1, B)
    gate_scale = jnp.concatenate(
        [jnp.full((2 * H, 1), 0.5), jnp.ones((H, 1)),
         jnp.full((H, 1), 0.5)]).astype(jnp.float32)
    W_ih = (W_ih * gate_scale).astype(jnp.bfloat16)
    W_hh = (W_hh * gate_scale).astype(jnp.bfloat16)
    bias = ((b_ih + b_hh).reshape(4 * H, 1) * gate_scale).astype(jnp.bfloat16)

    pooled = pl.pallas_call(
        _fused,
        out_shape=jax.ShapeDtypeStruct((H, B), jnp.float32),
    )(x, na, W_ih, W_hh, bias,
      Wpool1, bpool1.reshape(H, 1), Wself1, Wneigh1, b1.reshape(H, 1),
      Wpool2, bpool2.reshape(H, 1), Wself2, Wneigh2, b2.reshape(H, 1))

    return jnp.concatenate([pooled.T, hideout_obs, timestep_obs], axis=-1)
